# baseline (speedup 1.0000x reference)
import functools

import jax
import jax.numpy as jnp
from jax import lax
from jax.experimental import pallas as pl
from jax.experimental.pallas import tpu as pltpu
from jax.experimental.pallas import tpu_sc as plsc

MAX_DELTA = 256
NUM_ROWS = 2 * MAX_DELTA + 1
D_MODEL = 128
LANES = 16

NUM_CORES = 2
NUM_SUBCORES = 16
NUM_WORKERS = NUM_CORES * NUM_SUBCORES

CHUNK = 160
NGROUPS = CHUNK // LANES


def _make_sc_gather(total: int):
    assert total % (NUM_WORKERS * CHUNK * 2) == 0
    per_worker = total // NUM_WORKERS
    n_chunks = per_worker // CHUNK
    n_passes = n_chunks // 2
    mesh = plsc.VectorSubcoreMesh(core_axis_name="c", subcore_axis_name="s")

    @functools.partial(
        pl.kernel,
        out_type=jax.ShapeDtypeStruct((total, D_MODEL), jnp.float32),
        mesh=mesh,
        compiler_params=pltpu.CompilerParams(needs_layout_passes=False),
        scratch_types=[
            pltpu.VMEM((NUM_ROWS * D_MODEL,), jnp.float32),
            pltpu.VMEM((CHUNK,), jnp.int32),
            pltpu.VMEM((CHUNK,), jnp.int32),
            pltpu.VMEM((CHUNK, D_MODEL), jnp.float32),
            pltpu.VMEM((CHUNK, D_MODEL), jnp.float32),
            pltpu.SemaphoreType.DMA,
            pltpu.SemaphoreType.DMA,
            pltpu.SemaphoreType.DMA,
            pltpu.SemaphoreType.DMA,
            pltpu.SemaphoreType.DMA,
        ],
    )
    def sc_gather(delta_hbm, table_hbm, out_hbm, table_v, idx0_v, idx1_v,
                  rows0_v, rows1_v, sem_t, sem_i0, sem_i1, sem_o0, sem_o1):
        idxs = (idx0_v, idx1_v)
        rows = (rows0_v, rows1_v)
        sems_i = (sem_i0, sem_i1)
        sems_o = (sem_o0, sem_o1)
        wid = lax.axis_index("s") * NUM_CORES + lax.axis_index("c")
        base = wid * per_worker

        def start_idx(slot, c):
            pltpu.async_copy(
                delta_hbm.at[pl.ds(base + c * CHUNK, CHUNK)],
                idxs[slot], sems_i[slot])

        def wait_idx(slot):
            pltpu.make_async_copy(
                delta_hbm.at[pl.ds(0, CHUNK)],
                idxs[slot], sems_i[slot]).wait()

        def start_out(slot, c):
            pltpu.async_copy(
                rows[slot],
                out_hbm.at[pl.ds(base + c * CHUNK, CHUNK)], sems_o[slot])

        def wait_out(slot):
            pltpu.make_async_copy(
                rows[slot],
                out_hbm.at[pl.ds(0, CHUNK)], sems_o[slot]).wait()

        tbl = pltpu.async_copy(table_hbm, table_v, sem_t)
        start_idx(0, 0)
        start_idx(1, 1)
        tbl.wait()

        def fill_rows(slot):
            @plsc.parallel_loop(0, NGROUPS)
            def group(g):
                lane_iota = lax.iota(jnp.int32, LANES)
                sel = jnp.clip(
                    idxs[slot][pl.ds(g * LANES, LANES)] + MAX_DELTA,
                    0, 2 * MAX_DELTA)
                flat = sel * D_MODEL
                r16 = lane_iota + g * LANES

                @plsc.parallel_loop(0, D_MODEL, unroll=8)
                def col(c):
                    vals = plsc.load_gather(table_v, [flat + c])
                    plsc.store_scatter(
                        rows[slot], [r16, jnp.full((LANES,), c, jnp.int32)],
                        vals)

        def pass_body(g, carry):
            for b in range(2):
                c = g * 2 + b
                wait_idx(b)

                @pl.when(g > 0)
                def _():
                    wait_out(b)

                fill_rows(b)
                start_out(b, c)

                @pl.when(g + 1 < n_passes)
                def _():
                    start_idx(b, c + 2)
            return carry

        lax.fori_loop(0, n_passes, pass_body, 0)

        wait_out(0)
        wait_out(1)

    return sc_gather


def kernel(delta, table):
    total = delta.size
    flat = delta.reshape(total)
    out = _make_sc_gather(total)(flat, table.reshape(NUM_ROWS * D_MODEL))
    return out.reshape(*delta.shape, D_MODEL)


# speedup vs baseline: 6.2203x; 6.2203x over previous
import functools

import jax
import jax.numpy as jnp
from jax import lax
from jax.experimental import pallas as pl
from jax.experimental.pallas import tpu as pltpu
from jax.experimental.pallas import tpu_sc as plsc

MAX_DELTA = 256
NUM_ROWS = 2 * MAX_DELTA + 1
D_MODEL = 128
LANES = 16

NUM_CORES = 2
NUM_SUBCORES = 16
NUM_WORKERS = NUM_CORES * NUM_SUBCORES

CHUNK = 160
NGROUPS = CHUNK // LANES


def _make_sc_gather(total: int):
    assert total % (NUM_WORKERS * CHUNK * 2) == 0
    per_worker = total // NUM_WORKERS
    n_chunks = per_worker // CHUNK
    n_passes = n_chunks // 2
    mesh = plsc.VectorSubcoreMesh(core_axis_name="c", subcore_axis_name="s")

    @functools.partial(
        pl.kernel,
        out_type=jax.ShapeDtypeStruct((total, D_MODEL), jnp.float32),
        mesh=mesh,
        compiler_params=pltpu.CompilerParams(needs_layout_passes=False),
        scratch_types=[
            pltpu.VMEM((NUM_ROWS * D_MODEL,), jnp.float32),
            pltpu.VMEM((CHUNK,), jnp.int32),
            pltpu.VMEM((CHUNK,), jnp.int32),
            pltpu.VMEM((CHUNK, D_MODEL), jnp.float32),
            pltpu.VMEM((CHUNK, D_MODEL), jnp.float32),
            pltpu.SemaphoreType.DMA,
            pltpu.SemaphoreType.DMA,
            pltpu.SemaphoreType.DMA,
            pltpu.SemaphoreType.DMA,
            pltpu.SemaphoreType.DMA,
        ],
    )
    def sc_gather(delta_hbm, table_hbm, out_hbm, table_v, idx0_v, idx1_v,
                  rows0_v, rows1_v, sem_t, sem_i0, sem_i1, sem_o0, sem_o1):
        idxs = (idx0_v, idx1_v)
        rows = (rows0_v, rows1_v)
        sems_i = (sem_i0, sem_i1)
        sems_o = (sem_o0, sem_o1)
        wid = lax.axis_index("s") * NUM_CORES + lax.axis_index("c")
        base = wid * per_worker

        def start_idx(slot, c):
            pltpu.async_copy(
                delta_hbm.at[pl.ds(base + c * CHUNK, CHUNK)],
                idxs[slot], sems_i[slot])

        def wait_idx(slot):
            pltpu.make_async_copy(
                delta_hbm.at[pl.ds(0, CHUNK)],
                idxs[slot], sems_i[slot]).wait()

        def start_out(slot, c):
            pltpu.async_copy(
                rows[slot],
                out_hbm.at[pl.ds(base + c * CHUNK, CHUNK)], sems_o[slot])

        def wait_out(slot):
            pltpu.make_async_copy(
                rows[slot],
                out_hbm.at[pl.ds(0, CHUNK)], sems_o[slot]).wait()

        tbl = pltpu.async_copy(table_hbm, table_v, sem_t)
        start_idx(0, 0)
        start_idx(1, 1)
        tbl.wait()

        def fill_rows(slot):
            @plsc.parallel_loop(0, NGROUPS)
            def group(g):
                lane_iota = lax.iota(jnp.int32, LANES)
                sel = jnp.clip(
                    idxs[slot][pl.ds(g * LANES, LANES)] + MAX_DELTA,
                    0, 2 * MAX_DELTA)
                flat = sel * D_MODEL
                r16 = lane_iota + g * LANES

                @plsc.parallel_loop(0, D_MODEL, unroll=4)
                def col(c):
                    colv = (lane_iota + c) & (D_MODEL - 1)
                    vals = plsc.load_gather(table_v, [flat + colv])
                    plsc.store_scatter(rows[slot], [r16, colv], vals)

        def pass_body(g, carry):
            for b in range(2):
                c = g * 2 + b
                wait_idx(b)

                @pl.when(g > 0)
                def _():
                    wait_out(b)

                fill_rows(b)
                start_out(b, c)

                @pl.when(g + 1 < n_passes)
                def _():
                    start_idx(b, c + 2)
            return carry

        lax.fori_loop(0, n_passes, pass_body, 0)

        wait_out(0)
        wait_out(1)

    return sc_gather


def kernel(delta, table):
    total = delta.size
    flat = delta.reshape(total)
    out = _make_sc_gather(total)(flat, table.reshape(NUM_ROWS * D_MODEL))
    return out.reshape(*delta.shape, D_MODEL)


# speedup vs baseline: 7.0158x; 1.1279x over previous
import functools

import jax
import jax.numpy as jnp
from jax import lax
from jax.experimental import pallas as pl
from jax.experimental.pallas import tpu as pltpu
from jax.experimental.pallas import tpu_sc as plsc

MAX_DELTA = 256
NUM_ROWS = 2 * MAX_DELTA + 1
D_MODEL = 128
LANES = 16

NUM_CORES = 2
NUM_SUBCORES = 16
NUM_WORKERS = NUM_CORES * NUM_SUBCORES

CHUNK = 160
NGROUPS = CHUNK // LANES


def _make_sc_gather(total: int):
    assert total % (NUM_WORKERS * CHUNK * 2) == 0
    per_worker = total // NUM_WORKERS
    n_chunks = per_worker // CHUNK
    n_passes = n_chunks // 2
    mesh = plsc.VectorSubcoreMesh(core_axis_name="c", subcore_axis_name="s")

    @functools.partial(
        pl.kernel,
        out_type=jax.ShapeDtypeStruct((total, D_MODEL), jnp.float32),
        mesh=mesh,
        compiler_params=pltpu.CompilerParams(needs_layout_passes=False),
        scratch_types=[
            pltpu.VMEM((NUM_ROWS * D_MODEL,), jnp.float32),
            pltpu.VMEM((CHUNK,), jnp.int32),
            pltpu.VMEM((CHUNK,), jnp.int32),
            pltpu.VMEM((CHUNK, D_MODEL), jnp.float32),
            pltpu.VMEM((CHUNK, D_MODEL), jnp.float32),
            pltpu.SemaphoreType.DMA,
            pltpu.SemaphoreType.DMA,
            pltpu.SemaphoreType.DMA,
            pltpu.SemaphoreType.DMA,
            pltpu.SemaphoreType.DMA,
        ],
    )
    def sc_gather(delta_hbm, table_hbm, out_hbm, table_v, idx0_v, idx1_v,
                  rows0_v, rows1_v, sem_t, sem_i0, sem_i1, sem_o0, sem_o1):
        idxs = (idx0_v, idx1_v)
        rows = (rows0_v, rows1_v)
        sems_i = (sem_i0, sem_i1)
        sems_o = (sem_o0, sem_o1)
        wid = lax.axis_index("s") * NUM_CORES + lax.axis_index("c")
        base = wid * per_worker

        def start_idx(slot, c):
            pltpu.async_copy(
                delta_hbm.at[pl.ds(base + c * CHUNK, CHUNK)],
                idxs[slot], sems_i[slot])

        def wait_idx(slot):
            pltpu.make_async_copy(
                delta_hbm.at[pl.ds(0, CHUNK)],
                idxs[slot], sems_i[slot]).wait()

        def start_out(slot, c):
            pltpu.async_copy(
                rows[slot],
                out_hbm.at[pl.ds(base + c * CHUNK, CHUNK)], sems_o[slot])

        def wait_out(slot):
            pltpu.make_async_copy(
                rows[slot],
                out_hbm.at[pl.ds(0, CHUNK)], sems_o[slot]).wait()

        tbl = pltpu.async_copy(table_hbm, table_v, sem_t)
        start_idx(0, 0)
        start_idx(1, 1)
        tbl.wait()

        def fill_rows(slot):
            @plsc.parallel_loop(0, NGROUPS)
            def group(g):
                lane_iota = lax.iota(jnp.int32, LANES)
                sel = jnp.clip(
                    idxs[slot][pl.ds(g * LANES, LANES)] + MAX_DELTA,
                    0, 2 * MAX_DELTA)
                flat = sel * D_MODEL
                r16 = lane_iota + g * LANES

                @plsc.parallel_loop(0, D_MODEL, unroll=8)
                def col(c):
                    colv = (lane_iota + c) & (D_MODEL - 1)
                    vals = plsc.load_gather(table_v, [flat + colv])
                    plsc.store_scatter(rows[slot], [r16, colv], vals)

        def pass_body(g, carry):
            for b in range(2):
                c = g * 2 + b
                wait_idx(b)

                @pl.when(g > 0)
                def _():
                    wait_out(b)

                fill_rows(b)
                start_out(b, c)

                @pl.when(g + 1 < n_passes)
                def _():
                    start_idx(b, c + 2)
            return carry

        lax.fori_loop(0, n_passes, pass_body, 0)

        wait_out(0)
        wait_out(1)

    return sc_gather


def kernel(delta, table):
    total = delta.size
    flat = delta.reshape(total)
    out = _make_sc_gather(total)(flat, table.reshape(NUM_ROWS * D_MODEL))
    return out.reshape(*delta.shape, D_MODEL)


# speedup vs baseline: 7.5923x; 1.0822x over previous
import functools

import jax
import jax.numpy as jnp
from jax import lax
from jax.experimental import pallas as pl
from jax.experimental.pallas import tpu as pltpu
from jax.experimental.pallas import tpu_sc as plsc

MAX_DELTA = 256
NUM_ROWS = 2 * MAX_DELTA + 1
D_MODEL = 128
LANES = 16

NUM_CORES = 2
NUM_SUBCORES = 16
NUM_WORKERS = NUM_CORES * NUM_SUBCORES

CHUNK = 160
NGROUPS = CHUNK // LANES


def _make_sc_gather(total: int):
    assert total % (NUM_WORKERS * CHUNK * 2) == 0
    per_worker = total // NUM_WORKERS
    n_chunks = per_worker // CHUNK
    n_passes = n_chunks // 2
    mesh = plsc.VectorSubcoreMesh(core_axis_name="c", subcore_axis_name="s")

    @functools.partial(
        pl.kernel,
        out_type=jax.ShapeDtypeStruct((total, D_MODEL), jnp.float32),
        mesh=mesh,
        compiler_params=pltpu.CompilerParams(needs_layout_passes=False),
        scratch_types=[
            pltpu.VMEM((NUM_ROWS * D_MODEL,), jnp.float32),
            pltpu.VMEM((CHUNK,), jnp.int32),
            pltpu.VMEM((CHUNK,), jnp.int32),
            pltpu.VMEM((CHUNK, D_MODEL), jnp.float32),
            pltpu.VMEM((CHUNK, D_MODEL), jnp.float32),
            pltpu.SemaphoreType.DMA,
            pltpu.SemaphoreType.DMA,
            pltpu.SemaphoreType.DMA,
            pltpu.SemaphoreType.DMA,
            pltpu.SemaphoreType.DMA,
        ],
    )
    def sc_gather(delta_hbm, table_hbm, out_hbm, table_v, idx0_v, idx1_v,
                  rows0_v, rows1_v, sem_t, sem_i0, sem_i1, sem_o0, sem_o1):
        idxs = (idx0_v, idx1_v)
        rows = (rows0_v, rows1_v)
        sems_i = (sem_i0, sem_i1)
        sems_o = (sem_o0, sem_o1)
        wid = lax.axis_index("s") * NUM_CORES + lax.axis_index("c")
        base = wid * per_worker

        def start_idx(slot, c):
            pltpu.async_copy(
                delta_hbm.at[pl.ds(base + c * CHUNK, CHUNK)],
                idxs[slot], sems_i[slot])

        def wait_idx(slot):
            pltpu.make_async_copy(
                delta_hbm.at[pl.ds(0, CHUNK)],
                idxs[slot], sems_i[slot]).wait()

        def start_out(slot, c):
            pltpu.async_copy(
                rows[slot],
                out_hbm.at[pl.ds(base + c * CHUNK, CHUNK)], sems_o[slot])

        def wait_out(slot):
            pltpu.make_async_copy(
                rows[slot],
                out_hbm.at[pl.ds(0, CHUNK)], sems_o[slot]).wait()

        tbl = pltpu.async_copy(table_hbm, table_v, sem_t)
        start_idx(0, 0)
        start_idx(1, 1)
        tbl.wait()

        def fill_rows(slot):
            @plsc.parallel_loop(0, NGROUPS)
            def group(g):
                lane_iota = lax.iota(jnp.int32, LANES)
                sel = jnp.clip(
                    idxs[slot][pl.ds(g * LANES, LANES)] + MAX_DELTA,
                    0, 2 * MAX_DELTA)
                flat = sel * D_MODEL
                r16 = lane_iota + g * LANES

                @plsc.parallel_loop(0, D_MODEL, unroll=16)
                def col(c):
                    colv = (lane_iota + c) & (D_MODEL - 1)
                    vals = plsc.load_gather(table_v, [flat + colv])
                    plsc.store_scatter(rows[slot], [r16, colv], vals)

        def pass_body(g, carry):
            for b in range(2):
                c = g * 2 + b
                wait_idx(b)

                @pl.when(g > 0)
                def _():
                    wait_out(b)

                fill_rows(b)
                start_out(b, c)

                @pl.when(g + 1 < n_passes)
                def _():
                    start_idx(b, c + 2)
            return carry

        lax.fori_loop(0, n_passes, pass_body, 0)

        wait_out(0)
        wait_out(1)

    return sc_gather


def kernel(delta, table):
    total = delta.size
    flat = delta.reshape(total)
    out = _make_sc_gather(total)(flat, table.reshape(NUM_ROWS * D_MODEL))
    return out.reshape(*delta.shape, D_MODEL)
